# baseline (speedup 1.0000x reference)
import functools
import math

import jax
import jax.numpy as jnp
from jax import lax
from jax.experimental import pallas as pl
from jax.experimental.pallas import tpu as pltpu
from jax.experimental.pallas import tpu_sc as plsc

N = 10000
D = 128
D_OUT = 40
ALPHA = 0.1
THETA = 0.5
EPS = 1e-5

NC = 2
NS = 16
CHUNK = 128
N_PAD = 10112
ZROWS = N_PAD // NS
FRAC0 = 0.78


def _mesh():
    return plsc.VectorSubcoreMesh(core_axis_name="c", subcore_axis_name="s")


def _zero_buf(buf, rows, width):
    def body(i, carry):
        for l in range(width // 16):
            buf[i, pl.ds(l * 16, 16)] = jnp.zeros((16,), jnp.float32)
        return carry
    lax.fori_loop(0, rows, body, 0)


def _zero_acc_slice(buf, acc_sh, zbase):
    for r in range(ZROWS // CHUNK):
        pltpu.sync_copy(buf, acc_sh.at[pl.ds(zbase + r * CHUNK, CHUNK)])
    rem = ZROWS % CHUNK
    if rem:
        pltpu.sync_copy(buf.at[pl.ds(0, rem)],
                        acc_sh.at[pl.ds(zbase + (ZROWS // CHUNK) * CHUNK, rem)])


def _unpack_chunk(packed_v, j, src_st, dst_st):
    for l in range(CHUNK // 16):
        v = packed_v[j, pl.ds(l * 16, 16)]
        src_st[pl.ds(l * 16, 16)] = lax.bitwise_and(v, jnp.int32(0xFFFF))
        dst_st[pl.ds(l * 16, 16)] = lax.shift_right_logical(v, jnp.int32(16))


@functools.partial(jax.jit, static_argnames=("ch",))
def _degree(idx_p, ch):
    @functools.partial(
        pl.kernel,
        out_type=jax.ShapeDtypeStruct((NC, N_PAD, D), jnp.float32),
        mesh=_mesh(),
        scratch_types=[
            pltpu.VMEM((ch, CHUNK), jnp.int32),
            pltpu.VMEM((CHUNK,), jnp.int32),
            pltpu.VMEM((CHUNK,), jnp.int32),
            pltpu.VMEM((CHUNK, D), jnp.float32),
            pltpu.VMEM_SHARED((N_PAD, D), jnp.float32),
        ],
    )
    def deg_kernel(idx_hbm, out_hbm, idx_v, src_st, dst_st, ones_v, acc_sh):
        c = lax.axis_index("c")
        s = lax.axis_index("s")
        _zero_buf(ones_v, CHUNK, D)
        zbase = s * ZROWS
        _zero_acc_slice(ones_v, acc_sh, zbase)
        plsc.subcore_barrier()

        def fill_ones(i, carry):
            for l in range(D // 16):
                ones_v[i, pl.ds(l * 16, 16)] = jnp.ones((16,), jnp.float32)
            return carry
        lax.fori_loop(0, CHUNK, fill_ones, 0)
        pltpu.sync_copy(idx_hbm.at[c, s], idx_v)

        def body(j, carry):
            _unpack_chunk(idx_v, j, src_st, dst_st)
            pltpu.sync_copy(ones_v, acc_sh.at[dst_st], add=True)
            return carry
        lax.fori_loop(0, ch, body, 0)
        plsc.subcore_barrier()
        pltpu.sync_copy(acc_sh.at[pl.ds(zbase, ZROWS)],
                        out_hbm.at[c, pl.ds(zbase, ZROWS)])

    return deg_kernel(idx_p)


@functools.partial(jax.jit, static_argnames=("ch", "ch0", "ch1"))
def _propagate(hs, idx_p, ch, ch0, ch1):
    @functools.partial(
        pl.kernel,
        out_type=jax.ShapeDtypeStruct((NC, N_PAD, D), jnp.float32),
        mesh=_mesh(),
        scratch_types=[
            pltpu.VMEM((ch, CHUNK), jnp.int32),
            pltpu.VMEM((CHUNK,), jnp.int32),
            pltpu.VMEM((CHUNK,), jnp.int32),
            pltpu.VMEM((CHUNK,), jnp.int32),
            pltpu.VMEM((CHUNK,), jnp.int32),
            pltpu.VMEM((CHUNK, D), jnp.float32),
            pltpu.VMEM((CHUNK, D), jnp.float32),
            pltpu.VMEM_SHARED((N_PAD, D), jnp.float32),
            pltpu.SemaphoreType.DMA,
            pltpu.SemaphoreType.DMA,
        ],
    )
    def prop_kernel(hs_hbm, idx_hbm, out_hbm, idx_v,
                    src_e, dst_e, src_o, dst_o,
                    rows0_v, rows1_v, acc_sh, sem0, sem1):
        c = lax.axis_index("c")
        s = lax.axis_index("s")
        nch = jnp.where(c == 0, ch0, ch1)
        pltpu.sync_copy(idx_hbm.at[c, s], idx_v)
        _unpack_chunk(idx_v, 0, src_e, dst_e)
        pltpu.async_copy(hs_hbm.at[src_e], rows0_v, sem0)
        _zero_buf(rows1_v, CHUNK, D)
        zbase = s * ZROWS
        _zero_acc_slice(rows1_v, acc_sh, zbase)
        plsc.subcore_barrier()

        def body(j2, carry):
            a = 2 * j2
            b = a + 1
            _unpack_chunk(idx_v, b, src_o, dst_o)
            pltpu.async_copy(hs_hbm.at[src_o], rows1_v, sem1)
            pltpu.make_async_copy(hs_hbm.at[src_e], rows0_v, sem0).wait()
            pltpu.sync_copy(rows0_v, acc_sh.at[dst_e], add=True)
            _unpack_chunk(idx_v, jnp.minimum(a + 2, nch - 1), src_e, dst_e)
            pltpu.async_copy(hs_hbm.at[src_e], rows0_v, sem0)
            pltpu.make_async_copy(hs_hbm.at[src_o], rows1_v, sem1).wait()
            pltpu.sync_copy(rows1_v, acc_sh.at[dst_o], add=True)
            return carry
        lax.fori_loop(0, nch // 2, body, 0)
        pltpu.make_async_copy(hs_hbm.at[src_e], rows0_v, sem0).wait()
        plsc.subcore_barrier()
        pltpu.sync_copy(acc_sh.at[pl.ds(zbase, ZROWS)],
                        out_hbm.at[c, pl.ds(zbase, ZROWS)])

    return prop_kernel(hs, idx_p)


def _fc1(x, w, b, degp):
    def body(x_ref, w_ref, b_ref, degp_ref, x0_ref, hs_ref, dis_ref):
        dp = degp_ref[...]
        deg = dp[0, :N, 0:1] + dp[1, :N, 0:1] + 1.0
        dis = lax.rsqrt(deg)
        h = lax.dot_general(x_ref[...], w_ref[...], (((1,), (1,)), ((), ())),
                            preferred_element_type=jnp.float32)
        h = jnp.maximum(h + b_ref[...], 0.0)
        x0_ref[...] = h
        hs_ref[...] = h * dis
        dis_ref[...] = dis

    return pl.pallas_call(
        body,
        out_shape=(jax.ShapeDtypeStruct((N, D), jnp.float32),
                   jax.ShapeDtypeStruct((N, D), jnp.float32),
                   jax.ShapeDtypeStruct((N, 1), jnp.float32)),
    )(x, w, b, degp)


def _layer(p, hs, x0, dis, wc, g, bb, beta, wf=None, bf=None):
    last = wf is not None

    def body(p_ref, hs_ref, x0_ref, dis_ref, w_ref, g_ref, bb_ref, *rest):
        pp = p_ref[...]
        dis_v = dis_ref[...]
        ax = (pp[0, :N] + pp[1, :N] + hs_ref[...]) * dis_v
        hp = (1.0 - ALPHA) * ax + ALPHA * x0_ref[...]
        t = (1.0 - beta) * hp + beta * lax.dot_general(
            hp, w_ref[...], (((1,), (0,)), ((), ())),
            preferred_element_type=jnp.float32)
        mu = jnp.mean(t, axis=0, keepdims=True)
        var = jnp.mean((t - mu) ** 2, axis=0, keepdims=True)
        h = (t - mu) * lax.rsqrt(var + EPS) * g_ref[...] + bb_ref[...]
        h = jnp.maximum(h, 0.0)
        if last:
            wf_ref, bf_ref, out_ref = rest
            out_ref[...] = lax.dot_general(
                h, wf_ref[...], (((1,), (1,)), ((), ())),
                preferred_element_type=jnp.float32) + bf_ref[...]
        else:
            (out_ref,) = rest
            out_ref[...] = h * dis_v

    if last:
        return pl.pallas_call(
            body,
            out_shape=jax.ShapeDtypeStruct((N, D_OUT), jnp.float32),
        )(p, hs, x0, dis, wc, g, bb, wf, bf)
    return pl.pallas_call(
        body,
        out_shape=jax.ShapeDtypeStruct((N, D), jnp.float32),
    )(p, hs, x0, dis, wc, g, bb)


def kernel(x, edge_index, W_fc1, b_fc1, W_c0, W_c1, W_c2,
           g0, bb0, g1, bb1, g2, bb2, W_fc2, b_fc2):
    e = edge_index.shape[1]
    src = edge_index[0].astype(jnp.int32)
    dst = edge_index[1].astype(jnp.int32)
    packed = lax.shift_left(dst, 16) | src
    trash = jnp.int32(N << 16)

    ch_s = -(-e // (NC * NS * CHUNK))
    e_pad = NC * NS * ch_s * CHUNK
    idx_p = jnp.concatenate(
        [packed, jnp.full((e_pad - e,), trash, jnp.int32)]
    ).reshape(NC, NS, ch_s, CHUNK)

    e0 = int(e * FRAC0)
    ch0 = -(-e0 // (NS * CHUNK))
    ch0 += ch0 % 2
    ch1 = -(-(e - e0) // (NS * CHUNK))
    ch1 += ch1 % 2
    ch_a = max(ch0, ch1)
    cap = NS * ch_a * CHUNK

    def _part(part):
        return jnp.concatenate(
            [part, jnp.full((cap - part.shape[0],), trash, jnp.int32)]
        ).reshape(NS, ch_a, CHUNK)
    idx_a = jnp.stack([_part(packed[:e0]), _part(packed[e0:])])

    degp = _degree(idx_p, ch=ch_s)
    x0, hs, dis = _fc1(x, W_fc1, b_fc1.reshape(1, D), degp)

    wcs = (W_c0, W_c1, W_c2)
    gs = (g0, g1, g2)
    bbs = (bb0, bb1, bb2)
    out = None
    for i in range(3):
        p = _propagate(hs, idx_a, ch=ch_a, ch0=ch0, ch1=ch1)
        beta = math.log(THETA / (i + 1.0) + 1.0)
        if i < 2:
            hs = _layer(p, hs, x0, dis, wcs[i], gs[i].reshape(1, D),
                        bbs[i].reshape(1, D), beta)
        else:
            out = _layer(p, hs, x0, dis, wcs[i], gs[i].reshape(1, D),
                         bbs[i].reshape(1, D), beta,
                         wf=W_fc2, bf=b_fc2.reshape(1, D_OUT))
    return out


# speedup vs baseline: 6.6437x; 6.6437x over previous
import functools
import math

import jax
import jax.numpy as jnp
from jax import lax
from jax.experimental import pallas as pl
from jax.experimental.pallas import tpu as pltpu
from jax.experimental.pallas import tpu_sc as plsc

N = 10000
D = 128
D_OUT = 40
ALPHA = 0.1
THETA = 0.5
EPS = 1e-5

NC = 2
NS = 16
CHUNK = 128
N_PAD = 10112
ZROWS = N_PAD // NS
FRAC0 = 0.78


def _mesh():
    return plsc.VectorSubcoreMesh(core_axis_name="c", subcore_axis_name="s")


def _zero_buf(buf, rows, width):
    def body(i, carry):
        for l in range(width // 16):
            buf[i, pl.ds(l * 16, 16)] = jnp.zeros((16,), jnp.float32)
        return carry
    lax.fori_loop(0, rows, body, 0)


def _zero_acc_slice(buf, acc_sh, zbase):
    for r in range(ZROWS // CHUNK):
        pltpu.sync_copy(buf, acc_sh.at[pl.ds(zbase + r * CHUNK, CHUNK)])
    rem = ZROWS % CHUNK
    if rem:
        pltpu.sync_copy(buf.at[pl.ds(0, rem)],
                        acc_sh.at[pl.ds(zbase + (ZROWS // CHUNK) * CHUNK, rem)])


def _unpack_chunk(packed_v, j, src_st, dst_st):
    for l in range(CHUNK // 16):
        v = packed_v[j, pl.ds(l * 16, 16)]
        src_st[pl.ds(l * 16, 16)] = lax.bitwise_and(v, jnp.int32(0xFFFF))
        dst_st[pl.ds(l * 16, 16)] = lax.shift_right_logical(v, jnp.int32(16))


@functools.partial(jax.jit, static_argnames=("ch",))
def _degree(idx_p, ch):
    @functools.partial(
        pl.kernel,
        out_type=jax.ShapeDtypeStruct((NC, N_PAD, D), jnp.float32),
        mesh=_mesh(),
        scratch_types=[
            pltpu.VMEM((ch, CHUNK), jnp.int32),
            pltpu.VMEM((CHUNK,), jnp.int32),
            pltpu.VMEM((CHUNK,), jnp.int32),
            pltpu.VMEM((CHUNK, D), jnp.float32),
            pltpu.VMEM_SHARED((N_PAD, D), jnp.float32),
        ],
    )
    def deg_kernel(idx_hbm, out_hbm, idx_v, src_st, dst_st, ones_v, acc_sh):
        c = lax.axis_index("c")
        s = lax.axis_index("s")
        _zero_buf(ones_v, CHUNK, D)
        zbase = s * ZROWS
        _zero_acc_slice(ones_v, acc_sh, zbase)
        plsc.subcore_barrier()

        def fill_ones(i, carry):
            for l in range(D // 16):
                ones_v[i, pl.ds(l * 16, 16)] = jnp.ones((16,), jnp.float32)
            return carry
        lax.fori_loop(0, CHUNK, fill_ones, 0)
        pltpu.sync_copy(idx_hbm.at[c, s], idx_v)

        def body(j, carry):
            _unpack_chunk(idx_v, j, src_st, dst_st)
            pltpu.sync_copy(ones_v, acc_sh.at[dst_st], add=True)
            return carry
        lax.fori_loop(0, ch, body, 0)
        plsc.subcore_barrier()
        pltpu.sync_copy(acc_sh.at[pl.ds(zbase, ZROWS)],
                        out_hbm.at[c, pl.ds(zbase, ZROWS)])

    return deg_kernel(idx_p)


@functools.partial(jax.jit, static_argnames=("ch", "ch0", "ch1"))
def _propagate(hs, idx_p, ch, ch0, ch1):
    @functools.partial(
        pl.kernel,
        out_type=jax.ShapeDtypeStruct((NC, N_PAD, D), jnp.float32),
        mesh=_mesh(),
        scratch_types=[
            pltpu.VMEM((ch, CHUNK), jnp.int32),
            pltpu.VMEM((CHUNK,), jnp.int32),
            pltpu.VMEM((CHUNK,), jnp.int32),
            pltpu.VMEM((CHUNK,), jnp.int32),
            pltpu.VMEM((CHUNK,), jnp.int32),
            pltpu.VMEM((CHUNK, D), jnp.float32),
            pltpu.VMEM((CHUNK, D), jnp.float32),
            pltpu.VMEM_SHARED((N_PAD, D), jnp.float32),
            pltpu.SemaphoreType.DMA,
            pltpu.SemaphoreType.DMA,
        ],
    )
    def prop_kernel(hs_hbm, idx_hbm, out_hbm, idx_v,
                    src_e, dst_e, src_o, dst_o,
                    rows0_v, rows1_v, acc_sh, sem0, sem1):
        c = lax.axis_index("c")
        s = lax.axis_index("s")
        pltpu.sync_copy(idx_hbm.at[c, s], idx_v)
        _unpack_chunk(idx_v, 0, src_e, dst_e)
        pltpu.async_copy(hs_hbm.at[src_e], rows0_v, sem0)
        _zero_buf(rows1_v, CHUNK, D)
        zbase = s * ZROWS
        _zero_acc_slice(rows1_v, acc_sh, zbase)
        plsc.subcore_barrier()

        def edge_loop(ch_c):
            def body(j2, carry):
                a = 2 * j2
                b = a + 1
                _unpack_chunk(idx_v, b, src_o, dst_o)
                pltpu.async_copy(hs_hbm.at[src_o], rows1_v, sem1)
                pltpu.make_async_copy(hs_hbm.at[src_e], rows0_v, sem0).wait()
                pltpu.sync_copy(rows0_v, acc_sh.at[dst_e], add=True)
                _unpack_chunk(idx_v, jnp.minimum(a + 2, ch_c - 1), src_e, dst_e)
                pltpu.async_copy(hs_hbm.at[src_e], rows0_v, sem0)
                pltpu.make_async_copy(hs_hbm.at[src_o], rows1_v, sem1).wait()
                pltpu.sync_copy(rows1_v, acc_sh.at[dst_o], add=True)
                return carry
            lax.fori_loop(0, ch_c // 2, body, 0)

        @pl.when(c == 0)
        def _():
            edge_loop(ch0)

        @pl.when(c == 1)
        def _():
            edge_loop(ch1)
        pltpu.make_async_copy(hs_hbm.at[src_e], rows0_v, sem0).wait()
        plsc.subcore_barrier()
        pltpu.sync_copy(acc_sh.at[pl.ds(zbase, ZROWS)],
                        out_hbm.at[c, pl.ds(zbase, ZROWS)])

    return prop_kernel(hs, idx_p)


def _fc1(x, w, b, degp):
    def body(x_ref, w_ref, b_ref, degp_ref, x0_ref, hs_ref, dis_ref):
        dp = degp_ref[...]
        deg = dp[0, :N, 0:1] + dp[1, :N, 0:1] + 1.0
        dis = lax.rsqrt(deg)
        h = lax.dot_general(x_ref[...], w_ref[...], (((1,), (1,)), ((), ())),
                            preferred_element_type=jnp.float32)
        h = jnp.maximum(h + b_ref[...], 0.0)
        x0_ref[...] = h
        hs_ref[...] = h * dis
        dis_ref[...] = dis

    return pl.pallas_call(
        body,
        out_shape=(jax.ShapeDtypeStruct((N, D), jnp.float32),
                   jax.ShapeDtypeStruct((N, D), jnp.float32),
                   jax.ShapeDtypeStruct((N, 1), jnp.float32)),
    )(x, w, b, degp)


def _layer(p, hs, x0, dis, wc, g, bb, beta, wf=None, bf=None):
    last = wf is not None

    def body(p_ref, hs_ref, x0_ref, dis_ref, w_ref, g_ref, bb_ref, *rest):
        pp = p_ref[...]
        dis_v = dis_ref[...]
        ax = (pp[0, :N] + pp[1, :N] + hs_ref[...]) * dis_v
        hp = (1.0 - ALPHA) * ax + ALPHA * x0_ref[...]
        t = (1.0 - beta) * hp + beta * lax.dot_general(
            hp, w_ref[...], (((1,), (0,)), ((), ())),
            preferred_element_type=jnp.float32)
        mu = jnp.mean(t, axis=0, keepdims=True)
        var = jnp.mean((t - mu) ** 2, axis=0, keepdims=True)
        h = (t - mu) * lax.rsqrt(var + EPS) * g_ref[...] + bb_ref[...]
        h = jnp.maximum(h, 0.0)
        if last:
            wf_ref, bf_ref, out_ref = rest
            out_ref[...] = lax.dot_general(
                h, wf_ref[...], (((1,), (1,)), ((), ())),
                preferred_element_type=jnp.float32) + bf_ref[...]
        else:
            (out_ref,) = rest
            out_ref[...] = h * dis_v

    if last:
        return pl.pallas_call(
            body,
            out_shape=jax.ShapeDtypeStruct((N, D_OUT), jnp.float32),
        )(p, hs, x0, dis, wc, g, bb, wf, bf)
    return pl.pallas_call(
        body,
        out_shape=jax.ShapeDtypeStruct((N, D), jnp.float32),
    )(p, hs, x0, dis, wc, g, bb)


def kernel(x, edge_index, W_fc1, b_fc1, W_c0, W_c1, W_c2,
           g0, bb0, g1, bb1, g2, bb2, W_fc2, b_fc2):
    e = edge_index.shape[1]
    src = edge_index[0].astype(jnp.int32)
    dst = edge_index[1].astype(jnp.int32)
    packed = lax.shift_left(dst, 16) | src
    trash = jnp.int32(N << 16)

    ch_s = -(-e // (NC * NS * CHUNK))
    e_pad = NC * NS * ch_s * CHUNK
    idx_p = jnp.concatenate(
        [packed, jnp.full((e_pad - e,), trash, jnp.int32)]
    ).reshape(NC, NS, ch_s, CHUNK)

    e0 = int(e * FRAC0)
    ch0 = -(-e0 // (NS * CHUNK))
    ch0 += ch0 % 2
    ch1 = -(-(e - e0) // (NS * CHUNK))
    ch1 += ch1 % 2
    ch_a = max(ch0, ch1)

    def _part(part, ch_c):
        capc = NS * ch_c * CHUNK
        arr = jnp.concatenate(
            [part, jnp.full((capc - part.shape[0],), trash, jnp.int32)]
        ).reshape(NS, ch_c, CHUNK)
        if ch_c < ch_a:
            arr = jnp.pad(arr, ((0, 0), (0, ch_a - ch_c), (0, 0)),
                          constant_values=trash)
        return arr
    idx_a = jnp.stack([_part(packed[:e0], ch0), _part(packed[e0:], ch1)])

    degp = _degree(idx_p, ch=ch_s)
    x0, hs, dis = _fc1(x, W_fc1, b_fc1.reshape(1, D), degp)

    wcs = (W_c0, W_c1, W_c2)
    gs = (g0, g1, g2)
    bbs = (bb0, bb1, bb2)
    out = None
    for i in range(3):
        p = _propagate(hs, idx_a, ch=ch_a, ch0=ch0, ch1=ch1)
        beta = math.log(THETA / (i + 1.0) + 1.0)
        if i < 2:
            hs = _layer(p, hs, x0, dis, wcs[i], gs[i].reshape(1, D),
                        bbs[i].reshape(1, D), beta)
        else:
            out = _layer(p, hs, x0, dis, wcs[i], gs[i].reshape(1, D),
                         bbs[i].reshape(1, D), beta,
                         wf=W_fc2, bf=b_fc2.reshape(1, D_OUT))
    return out


# speedup vs baseline: 8.3489x; 1.2567x over previous
import functools
import math

import jax
import jax.numpy as jnp
from jax import lax
from jax.experimental import pallas as pl
from jax.experimental.pallas import tpu as pltpu
from jax.experimental.pallas import tpu_sc as plsc

N = 10000
D = 128
D_OUT = 40
ALPHA = 0.1
THETA = 0.5
EPS = 1e-5

NC = 2
NS = 16
CHUNK = 112
N_PAD = 10112
ZROWS = N_PAD // NS
FRAC0 = 0.85


def _mesh():
    return plsc.VectorSubcoreMesh(core_axis_name="c", subcore_axis_name="s")


def _zero_buf(buf, rows, width):
    def body(i, carry):
        for l in range(width // 16):
            buf[i, pl.ds(l * 16, 16)] = jnp.zeros((16,), jnp.float32)
        return carry
    lax.fori_loop(0, rows, body, 0)


def _zero_acc_slice(buf, acc_sh, zbase):
    for r in range(ZROWS // CHUNK):
        pltpu.sync_copy(buf, acc_sh.at[pl.ds(zbase + r * CHUNK, CHUNK)])
    rem = ZROWS % CHUNK
    if rem:
        pltpu.sync_copy(buf.at[pl.ds(0, rem)],
                        acc_sh.at[pl.ds(zbase + (ZROWS // CHUNK) * CHUNK, rem)])


def _unpack_chunk(packed_v, j, src_st, dst_st):
    for l in range(CHUNK // 16):
        v = packed_v[j, pl.ds(l * 16, 16)]
        src_st[pl.ds(l * 16, 16)] = lax.bitwise_and(v, jnp.int32(0xFFFF))
        dst_st[pl.ds(l * 16, 16)] = lax.shift_right_logical(v, jnp.int32(16))


@functools.partial(jax.jit, static_argnames=("ch",))
def _degree(idx_p, ch):
    @functools.partial(
        pl.kernel,
        out_type=jax.ShapeDtypeStruct((NC, N_PAD, D), jnp.float32),
        mesh=_mesh(),
        scratch_types=[
            pltpu.VMEM((ch, CHUNK), jnp.int32),
            pltpu.VMEM((CHUNK,), jnp.int32),
            pltpu.VMEM((CHUNK,), jnp.int32),
            pltpu.VMEM((CHUNK, D), jnp.float32),
            pltpu.VMEM_SHARED((N_PAD, D), jnp.float32),
        ],
    )
    def deg_kernel(idx_hbm, out_hbm, idx_v, src_st, dst_st, ones_v, acc_sh):
        c = lax.axis_index("c")
        s = lax.axis_index("s")
        _zero_buf(ones_v, CHUNK, D)
        zbase = s * ZROWS
        _zero_acc_slice(ones_v, acc_sh, zbase)
        plsc.subcore_barrier()

        def fill_ones(i, carry):
            for l in range(D // 16):
                ones_v[i, pl.ds(l * 16, 16)] = jnp.ones((16,), jnp.float32)
            return carry
        lax.fori_loop(0, CHUNK, fill_ones, 0)
        pltpu.sync_copy(idx_hbm.at[c, s], idx_v)

        def body(j, carry):
            _unpack_chunk(idx_v, j, src_st, dst_st)
            pltpu.sync_copy(ones_v, acc_sh.at[dst_st], add=True)
            return carry
        lax.fori_loop(0, ch, body, 0)
        plsc.subcore_barrier()
        pltpu.sync_copy(acc_sh.at[pl.ds(zbase, ZROWS)],
                        out_hbm.at[c, pl.ds(zbase, ZROWS)])

    return deg_kernel(idx_p)


@functools.partial(jax.jit, static_argnames=("ch", "ch0", "ch1"))
def _propagate(hs, idx_p, ch, ch0, ch1):
    @functools.partial(
        pl.kernel,
        out_type=jax.ShapeDtypeStruct((NC, N_PAD, D), jnp.float32),
        mesh=_mesh(),
        scratch_types=[
            pltpu.VMEM((ch, CHUNK), jnp.int32),
            pltpu.VMEM((CHUNK,), jnp.int32),
            pltpu.VMEM((CHUNK,), jnp.int32),
            pltpu.VMEM((CHUNK,), jnp.int32),
            pltpu.VMEM((CHUNK,), jnp.int32),
            pltpu.VMEM((CHUNK, D), jnp.float32),
            pltpu.VMEM((CHUNK, D), jnp.float32),
            pltpu.VMEM_SHARED((N_PAD, D), jnp.float32),
            pltpu.SemaphoreType.DMA,
            pltpu.SemaphoreType.DMA,
        ],
    )
    def prop_kernel(hs_hbm, idx_hbm, out_hbm, idx_v,
                    src_e, dst_e, src_o, dst_o,
                    rows0_v, rows1_v, acc_sh, sem0, sem1):
        c = lax.axis_index("c")
        s = lax.axis_index("s")
        pltpu.sync_copy(idx_hbm.at[c, s], idx_v)
        _unpack_chunk(idx_v, 0, src_e, dst_e)
        pltpu.async_copy(hs_hbm.at[src_e], rows0_v, sem0)
        _zero_buf(rows1_v, CHUNK, D)
        zbase = s * ZROWS
        _zero_acc_slice(rows1_v, acc_sh, zbase)
        plsc.subcore_barrier()

        def edge_loop(ch_c):
            def body(j2, carry):
                a = 2 * j2
                b = a + 1
                _unpack_chunk(idx_v, b, src_o, dst_o)
                pltpu.async_copy(hs_hbm.at[src_o], rows1_v, sem1)
                pltpu.make_async_copy(hs_hbm.at[src_e], rows0_v, sem0).wait()
                pltpu.sync_copy(rows0_v, acc_sh.at[dst_e], add=True)
                _unpack_chunk(idx_v, jnp.minimum(a + 2, ch_c - 1), src_e, dst_e)
                pltpu.async_copy(hs_hbm.at[src_e], rows0_v, sem0)
                pltpu.make_async_copy(hs_hbm.at[src_o], rows1_v, sem1).wait()
                pltpu.sync_copy(rows1_v, acc_sh.at[dst_o], add=True)
                return carry
            lax.fori_loop(0, ch_c // 2, body, 0)

        @pl.when(c == 0)
        def _():
            edge_loop(ch0)

        @pl.when(c == 1)
        def _():
            edge_loop(ch1)
        pltpu.make_async_copy(hs_hbm.at[src_e], rows0_v, sem0).wait()
        plsc.subcore_barrier()
        pltpu.sync_copy(acc_sh.at[pl.ds(zbase, ZROWS)],
                        out_hbm.at[c, pl.ds(zbase, ZROWS)])

    return prop_kernel(hs, idx_p)


def _fc1(x, w, b, degp):
    def body(x_ref, w_ref, b_ref, degp_ref, x0_ref, hs_ref, dis_ref):
        dp = degp_ref[...]
        deg = dp[0, :N, 0:1] + dp[1, :N, 0:1] + 1.0
        dis = lax.rsqrt(deg)
        h = lax.dot_general(x_ref[...], w_ref[...], (((1,), (1,)), ((), ())),
                            preferred_element_type=jnp.float32)
        h = jnp.maximum(h + b_ref[...], 0.0)
        x0_ref[...] = h
        hs_ref[...] = h * dis
        dis_ref[...] = dis

    return pl.pallas_call(
        body,
        out_shape=(jax.ShapeDtypeStruct((N, D), jnp.float32),
                   jax.ShapeDtypeStruct((N, D), jnp.float32),
                   jax.ShapeDtypeStruct((N, 1), jnp.float32)),
    )(x, w, b, degp)


def _layer(p, hs, x0, dis, wc, g, bb, beta, wf=None, bf=None):
    last = wf is not None

    def body(p_ref, hs_ref, x0_ref, dis_ref, w_ref, g_ref, bb_ref, *rest):
        pp = p_ref[...]
        dis_v = dis_ref[...]
        ax = (pp[0, :N] + pp[1, :N] + hs_ref[...]) * dis_v
        hp = (1.0 - ALPHA) * ax + ALPHA * x0_ref[...]
        t = (1.0 - beta) * hp + beta * lax.dot_general(
            hp, w_ref[...], (((1,), (0,)), ((), ())),
            preferred_element_type=jnp.float32)
        mu = jnp.mean(t, axis=0, keepdims=True)
        var = jnp.mean((t - mu) ** 2, axis=0, keepdims=True)
        h = (t - mu) * lax.rsqrt(var + EPS) * g_ref[...] + bb_ref[...]
        h = jnp.maximum(h, 0.0)
        if last:
            wf_ref, bf_ref, out_ref = rest
            out_ref[...] = lax.dot_general(
                h, wf_ref[...], (((1,), (1,)), ((), ())),
                preferred_element_type=jnp.float32) + bf_ref[...]
        else:
            (out_ref,) = rest
            out_ref[...] = h * dis_v

    if last:
        return pl.pallas_call(
            body,
            out_shape=jax.ShapeDtypeStruct((N, D_OUT), jnp.float32),
        )(p, hs, x0, dis, wc, g, bb, wf, bf)
    return pl.pallas_call(
        body,
        out_shape=jax.ShapeDtypeStruct((N, D), jnp.float32),
    )(p, hs, x0, dis, wc, g, bb)


def kernel(x, edge_index, W_fc1, b_fc1, W_c0, W_c1, W_c2,
           g0, bb0, g1, bb1, g2, bb2, W_fc2, b_fc2):
    e = edge_index.shape[1]
    src = edge_index[0].astype(jnp.int32)
    dst = edge_index[1].astype(jnp.int32)
    packed = lax.shift_left(dst, 16) | src
    trash = jnp.int32(N << 16)

    ch_s = -(-e // (NC * NS * CHUNK))
    e_pad = NC * NS * ch_s * CHUNK
    idx_p = jnp.concatenate(
        [packed, jnp.full((e_pad - e,), trash, jnp.int32)]
    ).reshape(NC, NS, ch_s, CHUNK)

    e0 = int(e * FRAC0)
    ch0 = -(-e0 // (NS * CHUNK))
    ch0 += ch0 % 2
    ch1 = -(-(e - e0) // (NS * CHUNK))
    ch1 += ch1 % 2
    ch_a = max(ch0, ch1)

    def _part(part, ch_c):
        capc = NS * ch_c * CHUNK
        arr = jnp.concatenate(
            [part, jnp.full((capc - part.shape[0],), trash, jnp.int32)]
        ).reshape(NS, ch_c, CHUNK)
        if ch_c < ch_a:
            arr = jnp.pad(arr, ((0, 0), (0, ch_a - ch_c), (0, 0)),
                          constant_values=trash)
        return arr
    idx_a = jnp.stack([_part(packed[:e0], ch0), _part(packed[e0:], ch1)])

    degp = _degree(idx_p, ch=ch_s)
    x0, hs, dis = _fc1(x, W_fc1, b_fc1.reshape(1, D), degp)

    wcs = (W_c0, W_c1, W_c2)
    gs = (g0, g1, g2)
    bbs = (bb0, bb1, bb2)
    out = None
    for i in range(3):
        p = _propagate(hs, idx_a, ch=ch_a, ch0=ch0, ch1=ch1)
        beta = math.log(THETA / (i + 1.0) + 1.0)
        if i < 2:
            hs = _layer(p, hs, x0, dis, wcs[i], gs[i].reshape(1, D),
                        bbs[i].reshape(1, D), beta)
        else:
            out = _layer(p, hs, x0, dis, wcs[i], gs[i].reshape(1, D),
                         bbs[i].reshape(1, D), beta,
                         wf=W_fc2, bf=b_fc2.reshape(1, D_OUT))
    return out
